# Initial kernel scaffold; baseline (speedup 1.0000x reference)
import jax
import jax.numpy as jnp
from jax.experimental import pallas as pl


def kernel(node_feat, edge_index, edge_feat, msg_w1, msg_b1, msg_w2, msg_b2, att_w1, att_b1, att_w2, att_b2, gru_wih, gru_whh, gru_bih, gru_bhh):
    raise NotImplementedError("write your pallas kernel here")


# speedup vs baseline: 2.8589x; 2.8589x over previous
import functools

import jax
import jax.numpy as jnp
from jax import lax
from jax.experimental import pallas as pl
from jax.experimental.pallas import tpu as pltpu
from jax.experimental.pallas import tpu_sc as plsc

NC = 2
NS = 16
NW = NC * NS
LANES = 16
K = 80


def _sc_mesh():
    return plsc.VectorSubcoreMesh(
        core_axis_name="c", subcore_axis_name="s", num_cores=NC, num_subcores=NS
    )


def _gather_diff(node_feat, src, dst):
    N, D = node_feat.shape
    E = src.shape[0]
    EW = E // NW
    CH = EW // K

    @functools.partial(
        pl.kernel,
        out_type=jax.ShapeDtypeStruct((E, D), jnp.float32),
        mesh=_sc_mesh(),
        scratch_types=[
            pltpu.VMEM((K,), jnp.int32),
            pltpu.VMEM((K,), jnp.int32),
            pltpu.VMEM((K, D), jnp.float32),
            pltpu.VMEM((K, D), jnp.float32),
            pltpu.SemaphoreType.DMA,
            pltpu.SemaphoreType.DMA,
        ],
    )
    def gather_k(node_hbm, src_hbm, dst_hbm, out_hbm,
                 idxs_v, idxd_v, rows_s, rows_d, sem_s, sem_d):
        wid = lax.axis_index("s") * NC + lax.axis_index("c")
        w0 = wid * EW

        @pl.loop(0, CH)
        def _chunk(i):
            base = w0 + i * K
            pltpu.sync_copy(src_hbm.at[pl.ds(base, K)], idxs_v)
            pltpu.sync_copy(dst_hbm.at[pl.ds(base, K)], idxd_v)
            cs = pltpu.async_copy(node_hbm.at[idxs_v], rows_s, sem_s)
            cd = pltpu.async_copy(node_hbm.at[idxd_v], rows_d, sem_d)
            cs.wait()
            cd.wait()

            @pl.loop(0, K)
            def _row(r):
                for j in range(D // LANES):
                    sl = pl.ds(j * LANES, LANES)
                    rows_s[r, sl] = rows_s[r, sl] - rows_d[r, sl]

            pltpu.sync_copy(rows_s, out_hbm.at[pl.ds(base, K)])

    return gather_k(node_feat, src, dst)


def _scatter_add(msg, dst, N):
    E, D = msg.shape
    EW = E // NW
    CH = EW // K
    zeros = jnp.zeros((N, D), jnp.float32)

    @functools.partial(
        pl.kernel,
        out_type=jax.ShapeDtypeStruct((NC, N, D), jnp.float32),
        mesh=_sc_mesh(),
        scratch_types=[
            pltpu.VMEM((K,), jnp.int32),
            pltpu.VMEM((K, D), jnp.float32),
            pltpu.VMEM_SHARED((N, D), jnp.float32),
            pltpu.SemaphoreType.DMA,
        ],
    )
    def scatter_k(msg_hbm, dst_hbm, zeros_hbm, out_hbm, idx_v, msg_v, acc_sh, sem):
        c = lax.axis_index("c")
        s = lax.axis_index("s")
        wid = s * NC + c

        @pl.when(s == 0)
        def _():
            pltpu.sync_copy(zeros_hbm, acc_sh)

        plsc.subcore_barrier()

        @pl.loop(0, CH)
        def _chunk(i):
            base = wid * EW + i * K
            pltpu.sync_copy(dst_hbm.at[pl.ds(base, K)], idx_v)
            pltpu.sync_copy(msg_hbm.at[pl.ds(base, K)], msg_v)
            pltpu.sync_copy(msg_v, acc_sh.at[idx_v], add=True)

        plsc.subcore_barrier()

        @pl.when(s == 0)
        def _():
            pltpu.sync_copy(acc_sh, out_hbm.at[c])

    return scatter_k(msg, dst, zeros)


def _edge_mlp(diff, ef, w1a, w1b, b1, w2, b2, aw1a, aw1b, ab1, aw2, ab2):
    E, D = diff.shape
    DE = ef.shape[1]
    MSG = w2.shape[1]
    B = 2560
    grid = E // B

    def body(diff_ref, ef_ref, w1a_ref, w1b_ref, b1_ref, w2_ref, b2_ref,
             aw1a_ref, aw1b_ref, ab1_ref, aw2_ref, ab2_ref, out_ref):
        x = diff_ref[...]
        f = ef_ref[...]
        t1 = jnp.dot(x, w1a_ref[...], preferred_element_type=jnp.float32)
        t1 = t1 + jnp.dot(f, w1b_ref[...], preferred_element_type=jnp.float32)
        h1 = jnp.maximum(t1 + b1_ref[...], 0.0)
        msg = jnp.dot(h1, w2_ref[...], preferred_element_type=jnp.float32) + b2_ref[...]
        a1 = jnp.dot(x, aw1a_ref[...], preferred_element_type=jnp.float32)
        a1 = a1 + jnp.dot(f, aw1b_ref[...], preferred_element_type=jnp.float32)
        a1 = jnp.maximum(a1 + ab1_ref[...], 0.0)
        att = jax.nn.sigmoid(
            jnp.dot(a1, aw2_ref[...], preferred_element_type=jnp.float32) + ab2_ref[...])
        out_ref[...] = msg * att

    full = lambda shape: pl.BlockSpec(shape, lambda i: (0, 0))
    return pl.pallas_call(
        body,
        grid=(grid,),
        in_specs=[
            pl.BlockSpec((B, D), lambda i: (i, 0)),
            pl.BlockSpec((B, DE), lambda i: (i, 0)),
            full((D, MSG)), full((DE, MSG)), full((1, MSG)),
            full((MSG, MSG)), full((1, MSG)),
            full((D, MSG)), full((DE, MSG)), full((1, MSG)),
            full((MSG, MSG)), full((1, MSG)),
        ],
        out_specs=pl.BlockSpec((B, MSG), lambda i: (i, 0)),
        out_shape=jax.ShapeDtypeStruct((E, MSG), jnp.float32),
    )(diff, ef, w1a, w1b, b1, w2, b2, aw1a, aw1b, ab1, aw2, ab2)


def _gru(parts, h, wih, whh, bih, bhh):
    _, N, D = parts.shape
    G = wih.shape[1]
    R = 2000
    grid = N // R

    def body(p_ref, h_ref, wih_ref, whh_ref, bih_ref, bhh_ref, out_ref):
        sm = p_ref[0] + p_ref[1]
        hh = h_ref[...]
        gi = jnp.dot(sm, wih_ref[...], preferred_element_type=jnp.float32) + bih_ref[...]
        gh = jnp.dot(hh, whh_ref[...], preferred_element_type=jnp.float32) + bhh_ref[...]
        i_r, i_z, i_n = gi[:, :D], gi[:, D:2 * D], gi[:, 2 * D:]
        h_r, h_z, h_n = gh[:, :D], gh[:, D:2 * D], gh[:, 2 * D:]
        r = jax.nn.sigmoid(i_r + h_r)
        z = jax.nn.sigmoid(i_z + h_z)
        n = jnp.tanh(i_n + r * h_n)
        out_ref[...] = (1.0 - z) * n + z * hh

    return pl.pallas_call(
        body,
        grid=(grid,),
        in_specs=[
            pl.BlockSpec((2, R, D), lambda i: (0, i, 0)),
            pl.BlockSpec((R, D), lambda i: (i, 0)),
            pl.BlockSpec((D, G), lambda i: (0, 0)),
            pl.BlockSpec((D, G), lambda i: (0, 0)),
            pl.BlockSpec((1, G), lambda i: (0, 0)),
            pl.BlockSpec((1, G), lambda i: (0, 0)),
        ],
        out_specs=pl.BlockSpec((R, D), lambda i: (i, 0)),
        out_shape=jax.ShapeDtypeStruct((N, D), jnp.float32),
    )(parts, h, wih, whh, bih, bhh)


def kernel(node_feat, edge_index, edge_feat, msg_w1, msg_b1, msg_w2, msg_b2,
           att_w1, att_b1, att_w2, att_b2, gru_wih, gru_whh, gru_bih, gru_bhh):
    N, D = node_feat.shape
    src = edge_index[0]
    dst = edge_index[1]

    diff = _gather_diff(node_feat, src, dst)

    msg = _edge_mlp(
        diff, edge_feat,
        msg_w1[:, :D].T, msg_w1[:, D:].T, msg_b1[None, :],
        msg_w2.T, msg_b2[None, :],
        att_w1[:, :D].T, att_w1[:, D:].T, att_b1[None, :],
        att_w2.T, att_b2[None, :],
    )

    parts = _scatter_add(msg, dst, N)

    return _gru(parts, node_feat, gru_wih.T, gru_whh.T,
                gru_bih[None, :], gru_bhh[None, :])
